# Initial kernel scaffold; baseline (speedup 1.0000x reference)
import functools

import jax
import jax.numpy as jnp
from jax import lax
from jax.experimental import pallas as pl
from jax.experimental.pallas import tpu as pltpu
from jax.experimental.pallas import tpu_sc as plsc

B, E, H = 8, 16384, 256
MR = MC = MT = 2048
TOTAL_ROWS = B * (MR * 6 + MC * 4 + MT * 4)
NW = 32
PER_W = TOTAL_ROWS // NW
CHUNK = 128
NCHUNK = PER_W // CHUNK


def _sc_gather(table, idx_all):
    mesh = plsc.VectorSubcoreMesh(core_axis_name="c", subcore_axis_name="s")

    @functools.partial(
        pl.kernel,
        mesh=mesh,
        out_type=jax.ShapeDtypeStruct((TOTAL_ROWS, H), jnp.float32),
        scratch_types=[
            pltpu.VMEM((PER_W,), jnp.int32),
            pltpu.VMEM((CHUNK, H), jnp.float32),
            pltpu.VMEM((CHUNK, H), jnp.float32),
            pltpu.SemaphoreType.DMA,
            pltpu.SemaphoreType.DMA,
        ],
    )
    def gather_kernel(table_hbm, idx_hbm, out_hbm, idx_v, buf0, buf1, sem0, sem1):
        wid = lax.axis_index("s") * 2 + lax.axis_index("c")
        base = wid * PER_W
        pltpu.sync_copy(idx_hbm.at[pl.ds(base, PER_W)], idx_v)

        bufs = (buf0, buf1)
        sems = (sem0, sem1)

        def issue(c, slot):
            pltpu.async_copy(
                table_hbm.at[idx_v.at[pl.ds(c * CHUNK, CHUNK)]], bufs[slot], sems[slot]
            )

        def drain(c, slot):
            pltpu.make_async_copy(
                table_hbm.at[idx_v.at[pl.ds(c * CHUNK, CHUNK)]], bufs[slot], sems[slot]
            ).wait()
            pltpu.sync_copy(bufs[slot], out_hbm.at[pl.ds(base + c * CHUNK, CHUNK)])

        issue(0, 0)

        def body(c, _):
            slot = lax.rem(c, 2)

            @pl.when(c + 1 < NCHUNK)
            def _issue_next():
                issue(c + 1, 1 - slot)

            drain(c, slot)
            return _

        lax.fori_loop(0, NCHUNK, body, 0)

    return gather_kernel(table, idx_all)


def _mlp_body(x_ref, w1_ref, b1_ref, w2_ref, b2_ref,
              wa1_ref, ba1_ref, wa2_ref, ba2_ref, wa3_ref, ba3_ref,
              wa4_ref, ba4_ref, out_ref):
    f32 = jnp.float32
    x = x_ref[...]
    h = jnp.maximum(jnp.dot(x, w1_ref[...], preferred_element_type=f32) + b1_ref[...], 0.0)
    m = jnp.dot(h, w2_ref[...], preferred_element_type=f32) + b2_ref[...]
    h = jnp.maximum(jnp.dot(m, wa1_ref[...], preferred_element_type=f32) + ba1_ref[...], 0.0)
    h = jnp.maximum(jnp.dot(h, wa2_ref[...], preferred_element_type=f32) + ba2_ref[...], 0.0)
    h = jnp.maximum(jnp.dot(h, wa3_ref[...], preferred_element_type=f32) + ba3_ref[...], 0.0)
    out_ref[...] = jnp.dot(h, wa4_ref[...], preferred_element_type=f32) + ba4_ref[...]


def _mlp_stack(x, w1, b1, w2, b2, wa1, ba1, wa2, ba2, wa3, ba3, wa4, ba4, row_block):
    n, k = x.shape
    grid = (n // row_block,)
    fixed = lambda i: (0, 0)
    out = pl.pallas_call(
        _mlp_body,
        grid=grid,
        in_specs=[
            pl.BlockSpec((row_block, k), lambda i: (i, 0)),
            pl.BlockSpec((k, H), fixed),
            pl.BlockSpec((1, H), fixed),
            pl.BlockSpec((H, H), fixed),
            pl.BlockSpec((1, H), fixed),
            pl.BlockSpec((H, H), fixed),
            pl.BlockSpec((1, H), fixed),
            pl.BlockSpec((H, H), fixed),
            pl.BlockSpec((1, H), fixed),
            pl.BlockSpec((H, H), fixed),
            pl.BlockSpec((1, H), fixed),
            pl.BlockSpec((H, 1), fixed),
            pl.BlockSpec((1, 1), fixed),
        ],
        out_specs=pl.BlockSpec((row_block, 1), lambda i: (i, 0)),
        out_shape=jax.ShapeDtypeStruct((n, 1), jnp.float32),
    )(x, w1, b1, w2, b2, wa1, ba1, wa2, ba2, wa3, ba3, wa4, ba4)
    return out


def kernel(e_emb, reloc_idx, cross_idx, twoopt_idx,
           Wr1, br1, Wr2, br2,
           Wc1, bc1, Wc2, bc2,
           Wa1, ba1, Wa2, ba2, Wa3, ba3, Wa4, ba4):
    offs = (jnp.arange(B, dtype=jnp.int32) * E)[:, None, None]
    ridx = (reloc_idx.astype(jnp.int32) + offs).reshape(-1)
    cidx = (cross_idx.astype(jnp.int32) + offs).reshape(-1)
    tidx = (twoopt_idx.astype(jnp.int32) + offs).reshape(-1)
    idx_all = jnp.concatenate([ridx, cidx, tidx])

    table = e_emb.reshape(B * E, H)
    gathered = _sc_gather(table, idx_all)

    n_r = B * MR * 6
    xr = gathered[:n_r].reshape(B * MR, 6 * H)
    xct = gathered[n_r:].reshape(B * (MC + MT), 4 * H)

    r1 = lambda v: v.reshape(1, -1)
    logits_r = _mlp_stack(xr, Wr1, r1(br1), Wr2, r1(br2),
                          Wa1, r1(ba1), Wa2, r1(ba2), Wa3, r1(ba3),
                          Wa4, r1(ba4), row_block=1024)
    logits_ct = _mlp_stack(xct, Wc1, r1(bc1), Wc2, r1(bc2),
                           Wa1, r1(ba1), Wa2, r1(ba2), Wa3, r1(ba3),
                           Wa4, r1(ba4), row_block=1024)

    lr = logits_r.reshape(B, MR)
    lc = logits_ct[: B * MC].reshape(B, MC)
    lt = logits_ct[B * MC:].reshape(B, MT)
    return jnp.concatenate([lr, lc, lt], axis=1)


# speedup vs baseline: 6.1778x; 6.1778x over previous
import functools

import jax
import jax.numpy as jnp
from jax import lax
from jax.experimental import pallas as pl
from jax.experimental.pallas import tpu as pltpu
from jax.experimental.pallas import tpu_sc as plsc

B, E, H = 8, 16384, 256
MR = MC = MT = 2048
TOTAL_ROWS = B * (MR * 6 + MC * 4 + MT * 4)
NW = 32
PER_W = TOTAL_ROWS // NW
CHUNK = 128
NCHUNK = PER_W // CHUNK


def _sc_gather(table, idx_all):
    mesh = plsc.VectorSubcoreMesh(core_axis_name="c", subcore_axis_name="s")

    @functools.partial(
        pl.kernel,
        mesh=mesh,
        out_type=jax.ShapeDtypeStruct((TOTAL_ROWS, H), jnp.float32),
        scratch_types=[
            pltpu.VMEM((PER_W,), jnp.int32),
            pltpu.VMEM((CHUNK, H), jnp.float32),
            pltpu.VMEM((CHUNK, H), jnp.float32),
            pltpu.SemaphoreType.DMA,
            pltpu.SemaphoreType.DMA,
        ],
    )
    def gather_kernel(table_hbm, idx_hbm, out_hbm, idx_v, buf0, buf1, sem0, sem1):
        wid = lax.axis_index("s") * 2 + lax.axis_index("c")
        base = wid * PER_W
        pltpu.sync_copy(idx_hbm.at[pl.ds(base, PER_W)], idx_v)

        def issue(c, buf, sem):
            pltpu.async_copy(
                table_hbm.at[idx_v.at[pl.ds(c * CHUNK, CHUNK)]], buf, sem
            )

        def drain(c, buf, sem):
            pltpu.make_async_copy(
                table_hbm.at[idx_v.at[pl.ds(c * CHUNK, CHUNK)]], buf, sem
            ).wait()
            pltpu.sync_copy(buf, out_hbm.at[pl.ds(base + c * CHUNK, CHUNK)])

        issue(0, buf0, sem0)

        def body(p, carry):
            c0 = p * 2
            issue(c0 + 1, buf1, sem1)
            drain(c0, buf0, sem0)

            @pl.when(c0 + 2 < NCHUNK)
            def _issue_next():
                issue(c0 + 2, buf0, sem0)

            drain(c0 + 1, buf1, sem1)
            return carry

        lax.fori_loop(0, NCHUNK // 2, body, 0)

    return gather_kernel(table, idx_all)


def _mlp_body(x_ref, w1_ref, b1_ref, w2_ref, b2_ref,
              wa1_ref, ba1_ref, wa2_ref, ba2_ref, wa3_ref, ba3_ref,
              wa4_ref, ba4_ref, out_ref):
    f32 = jnp.float32
    x = x_ref[...]
    h = jnp.maximum(jnp.dot(x, w1_ref[...], preferred_element_type=f32) + b1_ref[...], 0.0)
    m = jnp.dot(h, w2_ref[...], preferred_element_type=f32) + b2_ref[...]
    h = jnp.maximum(jnp.dot(m, wa1_ref[...], preferred_element_type=f32) + ba1_ref[...], 0.0)
    h = jnp.maximum(jnp.dot(h, wa2_ref[...], preferred_element_type=f32) + ba2_ref[...], 0.0)
    h = jnp.maximum(jnp.dot(h, wa3_ref[...], preferred_element_type=f32) + ba3_ref[...], 0.0)
    out_ref[...] = jnp.dot(h, wa4_ref[...], preferred_element_type=f32) + ba4_ref[...]


def _mlp_stack(x, w1, b1, w2, b2, wa1, ba1, wa2, ba2, wa3, ba3, wa4, ba4, row_block):
    n, k = x.shape
    grid = (n // row_block,)
    fixed = lambda i: (0, 0)
    out = pl.pallas_call(
        _mlp_body,
        grid=grid,
        in_specs=[
            pl.BlockSpec((row_block, k), lambda i: (i, 0)),
            pl.BlockSpec((k, H), fixed),
            pl.BlockSpec((1, H), fixed),
            pl.BlockSpec((H, H), fixed),
            pl.BlockSpec((1, H), fixed),
            pl.BlockSpec((H, H), fixed),
            pl.BlockSpec((1, H), fixed),
            pl.BlockSpec((H, H), fixed),
            pl.BlockSpec((1, H), fixed),
            pl.BlockSpec((H, H), fixed),
            pl.BlockSpec((1, H), fixed),
            pl.BlockSpec((H, 1), fixed),
            pl.BlockSpec((1, 1), fixed),
        ],
        out_specs=pl.BlockSpec((row_block, 1), lambda i: (i, 0)),
        out_shape=jax.ShapeDtypeStruct((n, 1), jnp.float32),
    )(x, w1, b1, w2, b2, wa1, ba1, wa2, ba2, wa3, ba3, wa4, ba4)
    return out


def kernel(e_emb, reloc_idx, cross_idx, twoopt_idx,
           Wr1, br1, Wr2, br2,
           Wc1, bc1, Wc2, bc2,
           Wa1, ba1, Wa2, ba2, Wa3, ba3, Wa4, ba4):
    offs = (jnp.arange(B, dtype=jnp.int32) * E)[:, None, None]
    ridx = (reloc_idx.astype(jnp.int32) + offs).reshape(-1)
    cidx = (cross_idx.astype(jnp.int32) + offs).reshape(-1)
    tidx = (twoopt_idx.astype(jnp.int32) + offs).reshape(-1)
    idx_all = jnp.concatenate([ridx, cidx, tidx])

    table = e_emb.reshape(B * E, H)
    gathered = _sc_gather(table, idx_all)

    n_r = B * MR * 6
    xr = gathered[:n_r].reshape(B * MR, 6 * H)
    xct = gathered[n_r:].reshape(B * (MC + MT), 4 * H)

    r1 = lambda v: v.reshape(1, -1)
    logits_r = _mlp_stack(xr, Wr1, r1(br1), Wr2, r1(br2),
                          Wa1, r1(ba1), Wa2, r1(ba2), Wa3, r1(ba3),
                          Wa4, r1(ba4), row_block=1024)
    logits_ct = _mlp_stack(xct, Wc1, r1(bc1), Wc2, r1(bc2),
                           Wa1, r1(ba1), Wa2, r1(ba2), Wa3, r1(ba3),
                           Wa4, r1(ba4), row_block=1024)

    lr = logits_r.reshape(B, MR)
    lc = logits_ct[: B * MC].reshape(B, MC)
    lt = logits_ct[B * MC:].reshape(B, MT)
    return jnp.concatenate([lr, lc, lt], axis=1)
